# Initial kernel scaffold; baseline (speedup 1.0000x reference)
import jax
import jax.numpy as jnp
from jax.experimental import pallas as pl


def kernel(x, W_gate, e_score_correction_bias):
    raise NotImplementedError("write your pallas kernel here")


# speedup vs baseline: 1.7445x; 1.7445x over previous
import functools

import jax
import jax.numpy as jnp
from jax.experimental import pallas as pl

DIM = 2048
NUM_EXPERTS = 64
TOP_K = 8
N_GROUPS = 8
TOPK_GROUP = 4
GROUP_SIZE = NUM_EXPERTS // N_GROUPS
ROUTED_SCALING_FACTOR = 2.5

_NEG = -1e30


def _router_block(x_ref, w_ref, b_ref, idx_ref, wgt_ref):
    x = x_ref[:]
    logits = jnp.dot(x, w_ref[:], preferred_element_type=jnp.float32)
    scores = jax.nn.sigmoid(logits)
    sfc = scores + b_ref[:]

    t = scores.shape[0]
    lane = jax.lax.broadcasted_iota(jnp.int32, (t, NUM_EXPERTS), 1)

    gsc = jnp.full((t, NUM_EXPERTS), _NEG, jnp.float32)
    for g in range(N_GROUPS):
        in_g = (lane >= g * GROUP_SIZE) & (lane < (g + 1) * GROUP_SIZE)
        vg = jnp.where(in_g, sfc, _NEG)
        m1 = jnp.max(vg, axis=1, keepdims=True)
        first = jnp.min(jnp.where(vg == m1, lane, NUM_EXPERTS), axis=1,
                        keepdims=True)
        m2 = jnp.max(jnp.where(lane == first, _NEG, vg), axis=1, keepdims=True)
        gsc = jnp.where(lane == g, m1 + m2, gsc)

    grp = lane // GROUP_SIZE
    expert_mask = jnp.zeros((t, NUM_EXPERTS), jnp.bool_)
    for _ in range(TOPK_GROUP):
        m = jnp.max(gsc, axis=1, keepdims=True)
        first = jnp.min(jnp.where(gsc == m, lane, NUM_EXPERTS), axis=1,
                        keepdims=True)
        expert_mask = expert_mask | (grp == first)
        gsc = jnp.where(lane == first, _NEG, gsc)

    tmp = jnp.where(expert_mask, sfc, 0.0)
    idx_cols = []
    wgt_cols = []
    for _ in range(TOP_K):
        m = jnp.max(tmp, axis=1, keepdims=True)
        first = jnp.min(jnp.where(tmp == m, lane, NUM_EXPERTS), axis=1,
                        keepdims=True)
        onehot = lane == first
        w = jnp.sum(jnp.where(onehot, scores, 0.0), axis=1, keepdims=True)
        idx_cols.append(first)
        wgt_cols.append(w)
        tmp = jnp.where(onehot, _NEG, tmp)

    idx = jnp.concatenate(idx_cols, axis=1)
    wgt = jnp.concatenate(wgt_cols, axis=1)
    denom = jnp.sum(wgt, axis=1, keepdims=True) + 1e-20
    wgt = wgt * (ROUTED_SCALING_FACTOR / denom)

    idx_ref[:] = idx
    wgt_ref[:] = wgt


@functools.partial(jax.jit, static_argnames=("block_t",))
def _run(x, w_t, bias, block_t=512):
    n = x.shape[0]
    grid = (n // block_t,)
    return pl.pallas_call(
        _router_block,
        grid=grid,
        in_specs=[
            pl.BlockSpec((block_t, DIM), lambda i: (i, 0)),
            pl.BlockSpec((DIM, NUM_EXPERTS), lambda i: (0, 0)),
            pl.BlockSpec((1, NUM_EXPERTS), lambda i: (0, 0)),
        ],
        out_specs=[
            pl.BlockSpec((block_t, TOP_K), lambda i: (i, 0)),
            pl.BlockSpec((block_t, TOP_K), lambda i: (i, 0)),
        ],
        out_shape=[
            jax.ShapeDtypeStruct((n, TOP_K), jnp.int32),
            jax.ShapeDtypeStruct((n, TOP_K), jnp.float32),
        ],
    )(x, w_t, bias)


def kernel(x, W_gate, e_score_correction_bias):
    w_t = W_gate.T
    bias = e_score_correction_bias.reshape(1, NUM_EXPERTS)
    idx, wgt = _run(x, w_t, bias)
    return idx, wgt


# speedup vs baseline: 5.4469x; 3.1223x over previous
import functools

import jax
import jax.numpy as jnp
import numpy as np
from jax.experimental import pallas as pl

DIM = 2048
NUM_EXPERTS = 64
TOP_K = 8
N_GROUPS = 8
TOPK_GROUP = 4
GROUP_SIZE = NUM_EXPERTS // N_GROUPS
ROUTED_SCALING_FACTOR = 2.5

_NEG = -1e30

_PERM = np.arange(NUM_EXPERTS).reshape(GROUP_SIZE, N_GROUPS).T.reshape(-1)


def _router_block(x_ref, w_ref, b_ref, idx_ref, wgt_ref):
    logits = jnp.dot(x_ref[:], w_ref[:], preferred_element_type=jnp.float32)
    lp = logits.T
    scores = jax.nn.sigmoid(lp)
    sfc = scores + b_ref[:]

    t = sfc.shape[1]

    m1 = sfc[0:N_GROUPS]
    m2 = jnp.full((N_GROUPS, t), _NEG, jnp.float32)
    for r in range(1, GROUP_SIZE):
        v = sfc[r * N_GROUPS:(r + 1) * N_GROUPS]
        m2 = jnp.maximum(m2, jnp.minimum(m1, v))
        m1 = jnp.maximum(m1, v)
    gsc = m1 + m2

    giota = jax.lax.broadcasted_iota(jnp.int32, (N_GROUPS, t), 0)
    sel = jnp.zeros((N_GROUPS, t), jnp.bool_)
    for _ in range(TOPK_GROUP):
        m = jnp.max(gsc, axis=0, keepdims=True)
        first = jnp.min(jnp.where(gsc == m, giota, N_GROUPS), axis=0,
                        keepdims=True)
        hit = giota == first
        sel = sel | hit
        gsc = jnp.where(hit, _NEG, gsc)

    tmp = jnp.concatenate(
        [jnp.where(sel, sfc[r * N_GROUPS:(r + 1) * N_GROUPS], 0.0)
         for r in range(GROUP_SIZE)], axis=0)

    srow = jax.lax.broadcasted_iota(jnp.int32, (NUM_EXPERTS, t), 0)
    eorig = ((srow << 3) & 56) | (srow >> 3)

    idx_rows = []
    wgt_rows = []
    for _ in range(TOP_K):
        m = jnp.max(tmp, axis=0, keepdims=True)
        first = jnp.min(jnp.where(tmp == m, eorig, NUM_EXPERTS), axis=0,
                        keepdims=True)
        onehot = eorig == first
        w = jnp.sum(jnp.where(onehot, scores, 0.0), axis=0, keepdims=True)
        idx_rows.append(first)
        wgt_rows.append(w)
        tmp = jnp.where(onehot, _NEG, tmp)

    idx = jnp.concatenate(idx_rows, axis=0)
    wgt = jnp.concatenate(wgt_rows, axis=0)
    denom = jnp.sum(wgt, axis=0, keepdims=True) + 1e-20
    wgt = wgt * (ROUTED_SCALING_FACTOR / denom)

    idx_ref[:] = idx.T
    wgt_ref[:] = wgt.T


@functools.partial(jax.jit, static_argnames=("block_t",))
def _run(x, w_t, bias, block_t=512):
    n = x.shape[0]
    grid = (n // block_t,)
    return pl.pallas_call(
        _router_block,
        grid=grid,
        in_specs=[
            pl.BlockSpec((block_t, DIM), lambda i: (i, 0)),
            pl.BlockSpec((DIM, NUM_EXPERTS), lambda i: (0, 0)),
            pl.BlockSpec((NUM_EXPERTS, 1), lambda i: (0, 0)),
        ],
        out_specs=[
            pl.BlockSpec((block_t, TOP_K), lambda i: (i, 0)),
            pl.BlockSpec((block_t, TOP_K), lambda i: (i, 0)),
        ],
        out_shape=[
            jax.ShapeDtypeStruct((n, TOP_K), jnp.int32),
            jax.ShapeDtypeStruct((n, TOP_K), jnp.float32),
        ],
    )(x, w_t, bias)


def kernel(x, W_gate, e_score_correction_bias):
    w_t = W_gate[_PERM].T
    bias = e_score_correction_bias[_PERM].reshape(NUM_EXPERTS, 1)
    idx, wgt = _run(x, w_t, bias)
    return idx, wgt


# speedup vs baseline: 6.3417x; 1.1643x over previous
import functools

import jax
import jax.numpy as jnp
import numpy as np
from jax.experimental import pallas as pl

DIM = 2048
NUM_EXPERTS = 64
TOP_K = 8
N_GROUPS = 8
TOPK_GROUP = 4
GROUP_SIZE = NUM_EXPERTS // N_GROUPS
ROUTED_SCALING_FACTOR = 2.5

_NEG = -1e30

_PERM = np.arange(NUM_EXPERTS).reshape(GROUP_SIZE, N_GROUPS).T.reshape(-1)


def _router_block(x_ref, w_ref, b_ref, idx_ref, wgt_ref):
    logits = jnp.dot(x_ref[:], w_ref[:], preferred_element_type=jnp.float32)
    lp = logits.T
    scores = jax.nn.sigmoid(lp)
    sfc = scores + b_ref[:]

    t = sfc.shape[1]

    m1 = sfc[0:N_GROUPS]
    m2 = jnp.full((N_GROUPS, t), _NEG, jnp.float32)
    for r in range(1, GROUP_SIZE):
        v = sfc[r * N_GROUPS:(r + 1) * N_GROUPS]
        m2 = jnp.maximum(m2, jnp.minimum(m1, v))
        m1 = jnp.maximum(m1, v)
    gsc = m1 + m2

    giota = jax.lax.broadcasted_iota(jnp.int32, (N_GROUPS, t), 0)
    sel = jnp.zeros((N_GROUPS, t), jnp.bool_)
    for _ in range(TOPK_GROUP):
        m = jnp.max(gsc, axis=0, keepdims=True)
        first = jnp.min(jnp.where(gsc == m, giota, N_GROUPS), axis=0,
                        keepdims=True)
        hit = giota == first
        sel = sel | hit
        gsc = jnp.where(hit, _NEG, gsc)

    tmp = jnp.concatenate(
        [jnp.where(sel, sfc[r * N_GROUPS:(r + 1) * N_GROUPS], 0.0)
         for r in range(GROUP_SIZE)], axis=0)

    srow = jax.lax.broadcasted_iota(jnp.int32, (NUM_EXPERTS, t), 0)
    eorig = ((srow << 3) & 56) | (srow >> 3)

    idx_rows = []
    wgt_rows = []
    for _ in range(TOP_K):
        m = jnp.max(tmp, axis=0, keepdims=True)
        first = jnp.min(jnp.where(tmp == m, eorig, NUM_EXPERTS), axis=0,
                        keepdims=True)
        onehot = eorig == first
        w = jnp.sum(jnp.where(onehot, scores, 0.0), axis=0, keepdims=True)
        idx_rows.append(first)
        wgt_rows.append(w)
        tmp = jnp.where(onehot, _NEG, tmp)

    idx = jnp.concatenate(idx_rows, axis=0)
    wgt = jnp.concatenate(wgt_rows, axis=0)
    denom = jnp.sum(wgt, axis=0, keepdims=True) + 1e-20
    wgt = wgt * (ROUTED_SCALING_FACTOR / denom)

    idx_ref[:] = idx.T
    wgt_ref[:] = wgt.T


@functools.partial(jax.jit, static_argnames=("block_t",))
def _run(x, w_t, bias, block_t=512):
    n = x.shape[0]
    grid = (n // block_t,)
    return pl.pallas_call(
        _router_block,
        grid=grid,
        in_specs=[
            pl.BlockSpec((block_t, DIM), lambda i: (i, 0)),
            pl.BlockSpec((DIM, NUM_EXPERTS), lambda i: (0, 0)),
            pl.BlockSpec((NUM_EXPERTS, 1), lambda i: (0, 0)),
        ],
        out_specs=[
            pl.BlockSpec((block_t, TOP_K), lambda i: (i, 0)),
            pl.BlockSpec((block_t, TOP_K), lambda i: (i, 0)),
        ],
        out_shape=[
            jax.ShapeDtypeStruct((n, TOP_K), jnp.int32),
            jax.ShapeDtypeStruct((n, TOP_K), jnp.float32),
        ],
    )(x, w_t, bias)


def kernel(x, W_gate, e_score_correction_bias):
    w_t = W_gate[_PERM].T
    bias = e_score_correction_bias[_PERM].reshape(NUM_EXPERTS, 1)
    idx, wgt = _run(x, w_t, bias, block_t=1024)
    return idx, wgt


# speedup vs baseline: 6.6867x; 1.0544x over previous
import functools

import jax
import jax.numpy as jnp
import numpy as np
from jax.experimental import pallas as pl

DIM = 2048
NUM_EXPERTS = 64
TOP_K = 8
N_GROUPS = 8
TOPK_GROUP = 4
GROUP_SIZE = NUM_EXPERTS // N_GROUPS
ROUTED_SCALING_FACTOR = 2.5

_NEG = -1e30

_PERM = np.arange(NUM_EXPERTS).reshape(GROUP_SIZE, N_GROUPS).T.reshape(-1)


def _router_block(x_ref, w_ref, b_ref, idx_ref, wgt_ref):
    logits = jnp.dot(x_ref[:], w_ref[:], preferred_element_type=jnp.float32)
    lp = logits.T
    scores = jax.nn.sigmoid(lp)
    sfc = scores + b_ref[:]

    t = sfc.shape[1]

    m1 = sfc[0:N_GROUPS]
    m2 = jnp.full((N_GROUPS, t), _NEG, jnp.float32)
    for r in range(1, GROUP_SIZE):
        v = sfc[r * N_GROUPS:(r + 1) * N_GROUPS]
        m2 = jnp.maximum(m2, jnp.minimum(m1, v))
        m1 = jnp.maximum(m1, v)
    gsc = m1 + m2

    giota = jax.lax.broadcasted_iota(jnp.int32, (N_GROUPS, t), 0)
    sel = jnp.zeros((N_GROUPS, t), jnp.bool_)
    for _ in range(TOPK_GROUP):
        m = jnp.max(gsc, axis=0, keepdims=True)
        first = jnp.min(jnp.where(gsc == m, giota, N_GROUPS), axis=0,
                        keepdims=True)
        hit = giota == first
        sel = sel | hit
        gsc = jnp.where(hit, _NEG, gsc)

    tmp = jnp.concatenate(
        [jnp.where(sel, sfc[r * N_GROUPS:(r + 1) * N_GROUPS], 0.0)
         for r in range(GROUP_SIZE)], axis=0)

    srow = jax.lax.broadcasted_iota(jnp.int32, (NUM_EXPERTS, t), 0)
    eorig = ((srow << 3) & 56) | (srow >> 3)

    idx_rows = []
    wgt_rows = []
    for _ in range(TOP_K):
        m = jnp.max(tmp, axis=0, keepdims=True)
        first = jnp.min(jnp.where(tmp == m, eorig, NUM_EXPERTS), axis=0,
                        keepdims=True)
        onehot = eorig == first
        w = jnp.sum(jnp.where(onehot, scores, 0.0), axis=0, keepdims=True)
        idx_rows.append(first)
        wgt_rows.append(w)
        tmp = jnp.where(onehot, _NEG, tmp)

    idx = jnp.concatenate(idx_rows, axis=0)
    wgt = jnp.concatenate(wgt_rows, axis=0)
    denom = jnp.sum(wgt, axis=0, keepdims=True) + 1e-20
    wgt = wgt * (ROUTED_SCALING_FACTOR / denom)

    idx_ref[:] = idx.T
    wgt_ref[:] = wgt.T


@functools.partial(jax.jit, static_argnames=("block_t",))
def _run(x, w_t, bias, block_t=512):
    n = x.shape[0]
    grid = (n // block_t,)
    return pl.pallas_call(
        _router_block,
        grid=grid,
        in_specs=[
            pl.BlockSpec((block_t, DIM), lambda i: (i, 0)),
            pl.BlockSpec((DIM, NUM_EXPERTS), lambda i: (0, 0)),
            pl.BlockSpec((NUM_EXPERTS, 1), lambda i: (0, 0)),
        ],
        out_specs=[
            pl.BlockSpec((block_t, TOP_K), lambda i: (i, 0)),
            pl.BlockSpec((block_t, TOP_K), lambda i: (i, 0)),
        ],
        out_shape=[
            jax.ShapeDtypeStruct((n, TOP_K), jnp.int32),
            jax.ShapeDtypeStruct((n, TOP_K), jnp.float32),
        ],
    )(x, w_t, bias)


def kernel(x, W_gate, e_score_correction_bias):
    w_t = W_gate[_PERM].T
    bias = e_score_correction_bias[_PERM].reshape(NUM_EXPERTS, 1)
    idx, wgt = _run(x, w_t, bias, block_t=2048)
    return idx, wgt
